# baseline (speedup 1.0000x reference)
import functools

import jax
import jax.numpy as jnp
from jax import lax
from jax.experimental import pallas as pl
from jax.experimental.pallas import tpu as pltpu
from jax.experimental.pallas import tpu_sc as plsc

N = 10000
NP = 10240
NSUB = 16
NCORE = 2
NTILE = NSUB * NCORE
CHUNK = 128
GRP = 16
RPT = NP // NSUB
ROWBLK = 256


@functools.lru_cache(maxsize=None)
def _make_agg(nch: int, npan: int, d: int):
    mesh = plsc.VectorSubcoreMesh(core_axis_name="c", subcore_axis_name="s")

    def body(src_hbm, dst_hbm, z_hbm, zeros_hbm, out_hbm,
             src_v, dst_v, g0, g1, acc, sem0, sem1, ssem0, ssem1):
        c = lax.axis_index("c")
        s = lax.axis_index("s")
        wid = c * NSUB + s

        for p in range(npan):
            zp = z_hbm.at[p]
            pltpu.sync_copy(zeros_hbm, acc.at[pl.ds(s * RPT, RPT)])
            plsc.subcore_barrier()

            def grp(gi, carry):
                pltpu.sync_copy(src_hbm.at[wid].at[pl.ds(gi * GRP, GRP)], src_v)
                pltpu.sync_copy(dst_hbm.at[wid].at[pl.ds(gi * GRP, GRP)], dst_v)
                pltpu.async_copy(zp.at[src_v.at[0]], g0, sem0)
                pltpu.async_copy(zp.at[src_v.at[1]], g1, sem1)

                def pair(jj, carry2):
                    j0 = 2 * jj
                    pltpu.make_async_copy(zp.at[src_v.at[j0]], g0, sem0).wait()
                    pltpu.async_copy(g0, acc.at[dst_v.at[j0]], ssem0, add=True)
                    pltpu.make_async_copy(zp.at[src_v.at[j0 + 1]], g1, sem1).wait()
                    pltpu.async_copy(g1, acc.at[dst_v.at[j0 + 1]], ssem1, add=True)
                    pltpu.make_async_copy(g0, acc.at[dst_v.at[j0]], ssem0).wait()
                    pltpu.make_async_copy(g1, acc.at[dst_v.at[j0 + 1]], ssem1).wait()

                    @pl.when(jj < GRP // 2 - 1)
                    def _():
                        pltpu.async_copy(zp.at[src_v.at[j0 + 2]], g0, sem0)
                        pltpu.async_copy(zp.at[src_v.at[j0 + 3]], g1, sem1)

                    return carry2

                lax.fori_loop(0, GRP // 2, pair, 0)
                return carry

            lax.fori_loop(0, nch // GRP, grp, 0)
            plsc.subcore_barrier()
            pltpu.sync_copy(acc.at[pl.ds(s * RPT, RPT)],
                            out_hbm.at[c].at[p].at[pl.ds(s * RPT, RPT)])

    return pl.kernel(
        body,
        mesh=mesh,
        compiler_params=pltpu.CompilerParams(use_tc_tiling_on_sc=False),
        out_type=jax.ShapeDtypeStruct((NCORE, npan, NP, d), jnp.float32),
        scratch_types=[
            pltpu.VMEM((GRP, CHUNK), jnp.int32),
            pltpu.VMEM((GRP, CHUNK), jnp.int32),
            pltpu.VMEM((CHUNK, d), jnp.float32),
            pltpu.VMEM((CHUNK, d), jnp.float32),
            pltpu.VMEM_SHARED((NP, d), jnp.float32),
            pltpu.SemaphoreType.DMA,
            pltpu.SemaphoreType.DMA,
            pltpu.SemaphoreType.DMA,
            pltpu.SemaphoreType.DMA,
        ],
    )


@functools.lru_cache(maxsize=None)
def _make_combine(d: int, relu: bool, post_scale: bool):
    def body(a_ref, u_ref, dv_ref, b_ref, o_ref):
        ssum = a_ref[0] + a_ref[1] + u_ref[...]
        y = dv_ref[...] * ssum + b_ref[...]
        if relu:
            y = jnp.maximum(y, 0.0)
        if post_scale:
            y = dv_ref[...] * y
        o_ref[...] = y

    return pl.pallas_call(
        body,
        grid=(NP // ROWBLK,),
        in_specs=[
            pl.BlockSpec((NCORE, ROWBLK, d), lambda i: (0, i, 0)),
            pl.BlockSpec((ROWBLK, d), lambda i: (i, 0)),
            pl.BlockSpec((ROWBLK, 1), lambda i: (i, 0)),
            pl.BlockSpec((1, d), lambda i: (0, 0)),
        ],
        out_specs=pl.BlockSpec((ROWBLK, d), lambda i: (i, 0)),
        out_shape=jax.ShapeDtypeStruct((NP, d), jnp.float32),
    )


@functools.lru_cache(maxsize=None)
def _make_matscale(k: int, d: int, relu: bool, scale: bool):
    def body(h_ref, w_ref, b_ref, dv_ref, o_ref):
        z = jnp.dot(h_ref[...], w_ref[...], preferred_element_type=jnp.float32)
        z = z + b_ref[...]
        if relu:
            z = jnp.maximum(z, 0.0)
        if scale:
            z = dv_ref[...] * z
        o_ref[...] = z

    return pl.pallas_call(
        body,
        grid=(NP // ROWBLK,),
        in_specs=[
            pl.BlockSpec((ROWBLK, k), lambda i: (i, 0)),
            pl.BlockSpec((k, d), lambda i: (0, 0)),
            pl.BlockSpec((1, d), lambda i: (0, 0)),
            pl.BlockSpec((ROWBLK, 1), lambda i: (i, 0)),
        ],
        out_specs=pl.BlockSpec((ROWBLK, d), lambda i: (i, 0)),
        out_shape=jax.ShapeDtypeStruct((NP, d), jnp.float32),
    )


def _dinv_body(dp_ref, o_ref):
    deg = dp_ref[0, :, 0:1] + dp_ref[1, :, 0:1] + 1.0
    o_ref[...] = lax.rsqrt(deg)


_dinv_call = None


def _make_dinv():
    global _dinv_call
    if _dinv_call is None:
        _dinv_call = pl.pallas_call(
            _dinv_body,
            grid=(NP // ROWBLK,),
            in_specs=[pl.BlockSpec((NCORE, ROWBLK, 16), lambda i: (0, i, 0))],
            out_specs=pl.BlockSpec((ROWBLK, 1), lambda i: (i, 0)),
            out_shape=jax.ShapeDtypeStruct((NP, 1), jnp.float32),
        )
    return _dinv_call


def kernel(x, edge_index, W0, b0, W1, b1, W2, b2, W3, b3, W4, b4, W5, b5,
           W6, b6, W7, b7):
    E = edge_index.shape[1]
    ept = -(-E // (NTILE * GRP * CHUNK)) * GRP * CHUNK
    nch = ept // CHUNK
    epad = ept * NTILE - E

    src = jnp.concatenate([edge_index[0], jnp.zeros((epad,), jnp.int32)])
    dst = jnp.concatenate([edge_index[1], jnp.full((epad,), NP - 1, jnp.int32)])
    src_r = src.reshape(NTILE, nch, CHUNK)
    dst_r = dst.reshape(NTILE, nch, CHUNK)

    def agg(z3, d):
        npan = z3.shape[0]
        zeros = jnp.zeros((RPT, d), jnp.float32)
        return _make_agg(nch, npan, d)(src_r, dst_r, z3, zeros)

    def agg1(z, d):
        return agg(z[None], d)[:, 0]

    def combine(a, u, dinv, b, d, relu, post_scale=False):
        bb = jnp.zeros((1, d), jnp.float32) if b is None else b.reshape(1, d)
        return _make_combine(d, relu, post_scale)(a, u, dinv, bb)

    def matscale(h, W, b, dinv, relu, scale):
        k, d = W.shape
        bb = jnp.zeros((1, d), jnp.float32) if b is None else b.reshape(1, d)
        return _make_matscale(k, d, relu, scale)(h, W, bb, dinv)

    ones16 = jnp.ones((NP, 16), jnp.float32)
    degp = agg1(ones16, 16)
    dinv = _make_dinv()(degp)

    xp = jnp.pad(x, ((0, NP - N), (0, 0)))

    u0 = matscale(xp, W0, None, dinv, relu=False, scale=True)
    a0 = agg1(u0, 64)
    h1 = combine(a0, u0, dinv, b0, 64, relu=True)
    u1 = matscale(h1, W1, None, dinv, relu=False, scale=True)
    a1 = agg1(u1, 64)
    h2 = combine(a1, u1, dinv, b1, 64, relu=True)
    u2 = matscale(h2, W2, None, dinv, relu=False, scale=True)
    a2 = agg1(u2, 64)
    u3 = combine(a2, u2, dinv, b2, 64, relu=True, post_scale=True)
    a3 = agg1(u3, 64)
    q3 = combine(a3, u3, dinv, None, 64, relu=False)
    u4 = matscale(q3, W3, b3, dinv, relu=True, scale=True)
    a4 = agg1(u4, 128)
    q4 = combine(a4, u4, dinv, None, 128, relu=False)
    h5 = matscale(q4, W4, b4, dinv, relu=True, scale=False)
    u5 = matscale(h5, W5, None, dinv, relu=False, scale=True)
    u5p3 = u5.reshape(NP, 4, 128).transpose(1, 0, 2)
    a5 = agg(u5p3, 128)
    h6p = [combine(a5[:, p], u5p3[p], dinv,
                   lax.slice_in_dim(b5, p * 128, (p + 1) * 128), 128, relu=True)
           for p in range(4)]
    h6 = jnp.concatenate(h6p, axis=1)
    u6 = matscale(h6, W6, None, dinv, relu=False, scale=True)
    u6p3 = u6.reshape(NP, 2, 128).transpose(1, 0, 2)
    a6 = agg(u6p3, 128)
    h7p = [combine(a6[:, p], u6p3[p], dinv,
                   lax.slice_in_dim(b6, p * 128, (p + 1) * 128), 128, relu=True)
           for p in range(2)]
    h7 = jnp.concatenate(h7p, axis=1)
    W7p = jnp.pad(W7, ((0, 0), (0, 14)))
    b7p = jnp.pad(b7, ((0, 14),))
    u7 = matscale(h7, W7p, None, dinv, relu=False, scale=True)
    a7 = agg1(u7, 16)
    outp = combine(a7, u7, dinv, b7p, 16, relu=False)
    return outp[:N, :2]


# speedup vs baseline: 1.0915x; 1.0915x over previous
import functools

import jax
import jax.numpy as jnp
from jax import lax
from jax.experimental import pallas as pl
from jax.experimental.pallas import tpu as pltpu
from jax.experimental.pallas import tpu_sc as plsc

N = 10000
NP = 10240
NSUB = 16
NCORE = 2
NTILE = NSUB * NCORE
CHUNK = 128
GRP = 16
RPT = NP // NSUB
ROWBLK = 256


@functools.lru_cache(maxsize=None)
def _make_agg(nch: int, npan: int, d: int):
    mesh = plsc.VectorSubcoreMesh(core_axis_name="c", subcore_axis_name="s")

    def body(src_hbm, dst_hbm, z_hbm, zeros_hbm, out_hbm,
             src_v, dst_v, g0, g1, acc, sem0, sem1, ssem0, ssem1):
        c = lax.axis_index("c")
        s = lax.axis_index("s")
        wid = c * NSUB + s

        for p in range(npan):
            zp = z_hbm.at[p]
            pltpu.sync_copy(zeros_hbm, acc.at[pl.ds(s * RPT, RPT)])
            plsc.subcore_barrier()

            def grp(gi, carry):
                pltpu.sync_copy(src_hbm.at[wid].at[pl.ds(gi * GRP, GRP)], src_v)
                pltpu.sync_copy(dst_hbm.at[wid].at[pl.ds(gi * GRP, GRP)], dst_v)
                pltpu.async_copy(zp.at[src_v.at[0]], g0, sem0)

                def pair(jj, carry2):
                    j0 = 2 * jj
                    pltpu.async_copy(zp.at[src_v.at[j0 + 1]], g1, sem1)
                    pltpu.make_async_copy(zp.at[src_v.at[j0]], g0, sem0).wait()
                    pltpu.sync_copy(g0, acc.at[dst_v.at[j0]], add=True)

                    @pl.when(jj < GRP // 2 - 1)
                    def _():
                        pltpu.async_copy(zp.at[src_v.at[j0 + 2]], g0, sem0)

                    pltpu.make_async_copy(zp.at[src_v.at[j0 + 1]], g1, sem1).wait()
                    pltpu.sync_copy(g1, acc.at[dst_v.at[j0 + 1]], add=True)
                    return carry2

                lax.fori_loop(0, GRP // 2, pair, 0)
                return carry

            lax.fori_loop(0, nch // GRP, grp, 0)
            plsc.subcore_barrier()
            pltpu.sync_copy(acc.at[pl.ds(s * RPT, RPT)],
                            out_hbm.at[c].at[p].at[pl.ds(s * RPT, RPT)])

    return pl.kernel(
        body,
        mesh=mesh,
        compiler_params=pltpu.CompilerParams(use_tc_tiling_on_sc=False),
        out_type=jax.ShapeDtypeStruct((NCORE, npan, NP, d), jnp.float32),
        scratch_types=[
            pltpu.VMEM((GRP, CHUNK), jnp.int32),
            pltpu.VMEM((GRP, CHUNK), jnp.int32),
            pltpu.VMEM((CHUNK, d), jnp.float32),
            pltpu.VMEM((CHUNK, d), jnp.float32),
            pltpu.VMEM_SHARED((NP, d), jnp.float32),
            pltpu.SemaphoreType.DMA,
            pltpu.SemaphoreType.DMA,
            pltpu.SemaphoreType.DMA,
            pltpu.SemaphoreType.DMA,
        ],
    )


@functools.lru_cache(maxsize=None)
def _make_combine(d: int, relu: bool, post_scale: bool):
    def body(a_ref, u_ref, dv_ref, b_ref, o_ref):
        ssum = a_ref[0] + a_ref[1] + u_ref[...]
        y = dv_ref[...] * ssum + b_ref[...]
        if relu:
            y = jnp.maximum(y, 0.0)
        if post_scale:
            y = dv_ref[...] * y
        o_ref[...] = y

    return pl.pallas_call(
        body,
        grid=(NP // ROWBLK,),
        in_specs=[
            pl.BlockSpec((NCORE, ROWBLK, d), lambda i: (0, i, 0)),
            pl.BlockSpec((ROWBLK, d), lambda i: (i, 0)),
            pl.BlockSpec((ROWBLK, 1), lambda i: (i, 0)),
            pl.BlockSpec((1, d), lambda i: (0, 0)),
        ],
        out_specs=pl.BlockSpec((ROWBLK, d), lambda i: (i, 0)),
        out_shape=jax.ShapeDtypeStruct((NP, d), jnp.float32),
    )


@functools.lru_cache(maxsize=None)
def _make_matscale(k: int, d: int, relu: bool, scale: bool):
    def body(h_ref, w_ref, b_ref, dv_ref, o_ref):
        z = jnp.dot(h_ref[...], w_ref[...], preferred_element_type=jnp.float32)
        z = z + b_ref[...]
        if relu:
            z = jnp.maximum(z, 0.0)
        if scale:
            z = dv_ref[...] * z
        o_ref[...] = z

    return pl.pallas_call(
        body,
        grid=(NP // ROWBLK,),
        in_specs=[
            pl.BlockSpec((ROWBLK, k), lambda i: (i, 0)),
            pl.BlockSpec((k, d), lambda i: (0, 0)),
            pl.BlockSpec((1, d), lambda i: (0, 0)),
            pl.BlockSpec((ROWBLK, 1), lambda i: (i, 0)),
        ],
        out_specs=pl.BlockSpec((ROWBLK, d), lambda i: (i, 0)),
        out_shape=jax.ShapeDtypeStruct((NP, d), jnp.float32),
    )


def _dinv_body(dp_ref, o_ref):
    deg = dp_ref[0, :, 0:1] + dp_ref[1, :, 0:1] + 1.0
    o_ref[...] = lax.rsqrt(deg)


_dinv_call = None


def _make_dinv():
    global _dinv_call
    if _dinv_call is None:
        _dinv_call = pl.pallas_call(
            _dinv_body,
            grid=(NP // ROWBLK,),
            in_specs=[pl.BlockSpec((NCORE, ROWBLK, 16), lambda i: (0, i, 0))],
            out_specs=pl.BlockSpec((ROWBLK, 1), lambda i: (i, 0)),
            out_shape=jax.ShapeDtypeStruct((NP, 1), jnp.float32),
        )
    return _dinv_call


def kernel(x, edge_index, W0, b0, W1, b1, W2, b2, W3, b3, W4, b4, W5, b5,
           W6, b6, W7, b7):
    E = edge_index.shape[1]
    ept = -(-E // (NTILE * GRP * CHUNK)) * GRP * CHUNK
    nch = ept // CHUNK
    epad = ept * NTILE - E

    src = jnp.concatenate([edge_index[0], jnp.zeros((epad,), jnp.int32)])
    dst = jnp.concatenate([edge_index[1], jnp.full((epad,), NP - 1, jnp.int32)])
    src_r = src.reshape(NTILE, nch, CHUNK)
    dst_r = dst.reshape(NTILE, nch, CHUNK)

    def agg(z3, d):
        npan = z3.shape[0]
        zeros = jnp.zeros((RPT, d), jnp.float32)
        return _make_agg(nch, npan, d)(src_r, dst_r, z3, zeros)

    def agg1(z, d):
        return agg(z[None], d)[:, 0]

    def combine(a, u, dinv, b, d, relu, post_scale=False):
        bb = jnp.zeros((1, d), jnp.float32) if b is None else b.reshape(1, d)
        return _make_combine(d, relu, post_scale)(a, u, dinv, bb)

    def matscale(h, W, b, dinv, relu, scale):
        k, d = W.shape
        bb = jnp.zeros((1, d), jnp.float32) if b is None else b.reshape(1, d)
        return _make_matscale(k, d, relu, scale)(h, W, bb, dinv)

    ones16 = jnp.ones((NP, 16), jnp.float32)
    degp = agg1(ones16, 16)
    dinv = _make_dinv()(degp)

    xp = jnp.pad(x, ((0, NP - N), (0, 0)))

    u0 = matscale(xp, W0, None, dinv, relu=False, scale=True)
    a0 = agg1(u0, 64)
    h1 = combine(a0, u0, dinv, b0, 64, relu=True)
    u1 = matscale(h1, W1, None, dinv, relu=False, scale=True)
    a1 = agg1(u1, 64)
    h2 = combine(a1, u1, dinv, b1, 64, relu=True)
    u2 = matscale(h2, W2, None, dinv, relu=False, scale=True)
    a2 = agg1(u2, 64)
    u3 = combine(a2, u2, dinv, b2, 64, relu=True, post_scale=True)
    a3 = agg1(u3, 64)
    q3 = combine(a3, u3, dinv, None, 64, relu=False)
    u4 = matscale(q3, W3, b3, dinv, relu=True, scale=True)
    a4 = agg1(u4, 128)
    q4 = combine(a4, u4, dinv, None, 128, relu=False)
    h5 = matscale(q4, W4, b4, dinv, relu=True, scale=False)
    u5 = matscale(h5, W5, None, dinv, relu=False, scale=True)
    u5p3 = u5.reshape(NP, 4, 128).transpose(1, 0, 2)
    a5 = agg(u5p3, 128)
    h6p = [combine(a5[:, p], u5p3[p], dinv,
                   lax.slice_in_dim(b5, p * 128, (p + 1) * 128), 128, relu=True)
           for p in range(4)]
    h6 = jnp.concatenate(h6p, axis=1)
    u6 = matscale(h6, W6, None, dinv, relu=False, scale=True)
    u6p3 = u6.reshape(NP, 2, 128).transpose(1, 0, 2)
    a6 = agg(u6p3, 128)
    h7p = [combine(a6[:, p], u6p3[p], dinv,
                   lax.slice_in_dim(b6, p * 128, (p + 1) * 128), 128, relu=True)
           for p in range(2)]
    h7 = jnp.concatenate(h7p, axis=1)
    W7p = jnp.pad(W7, ((0, 0), (0, 14)))
    b7p = jnp.pad(b7, ((0, 14),))
    u7 = matscale(h7, W7p, None, dinv, relu=False, scale=True)
    a7 = agg1(u7, 16)
    outp = combine(a7, u7, dinv, b7p, 16, relu=False)
    return outp[:N, :2]


# speedup vs baseline: 1.1081x; 1.0151x over previous
import functools

import jax
import jax.numpy as jnp
from jax import lax
from jax.experimental import pallas as pl
from jax.experimental.pallas import tpu as pltpu
from jax.experimental.pallas import tpu_sc as plsc

N = 10000
NP = 10240
NSUB = 16
NCORE = 2
NTILE = NSUB * NCORE
CHUNK = 128
GRP = 32
RPT = NP // NSUB
ROWBLK = 256


@functools.lru_cache(maxsize=None)
def _make_agg(nch: int, npan: int, d: int):
    mesh = plsc.VectorSubcoreMesh(core_axis_name="c", subcore_axis_name="s")

    def body(src_hbm, dst_hbm, z_hbm, zeros_hbm, out_hbm,
             src_v, dst_v, g0, g1, acc, sem0, sem1, ssem0, ssem1):
        c = lax.axis_index("c")
        s = lax.axis_index("s")
        wid = c * NSUB + s

        for p in range(npan):
            zp = z_hbm.at[p]
            pltpu.sync_copy(zeros_hbm, acc.at[pl.ds(s * RPT, RPT)])
            plsc.subcore_barrier()

            def grp(gi, carry):
                pltpu.sync_copy(src_hbm.at[wid].at[pl.ds(gi * GRP, GRP)], src_v)
                pltpu.sync_copy(dst_hbm.at[wid].at[pl.ds(gi * GRP, GRP)], dst_v)
                pltpu.async_copy(zp.at[src_v.at[0]], g0, sem0)

                def pair(jj, carry2):
                    j0 = 2 * jj
                    pltpu.async_copy(zp.at[src_v.at[j0 + 1]], g1, sem1)
                    pltpu.make_async_copy(zp.at[src_v.at[j0]], g0, sem0).wait()
                    pltpu.sync_copy(g0, acc.at[dst_v.at[j0]], add=True)

                    @pl.when(jj < GRP // 2 - 1)
                    def _():
                        pltpu.async_copy(zp.at[src_v.at[j0 + 2]], g0, sem0)

                    pltpu.make_async_copy(zp.at[src_v.at[j0 + 1]], g1, sem1).wait()
                    pltpu.sync_copy(g1, acc.at[dst_v.at[j0 + 1]], add=True)
                    return carry2

                lax.fori_loop(0, GRP // 2, pair, 0)
                return carry

            lax.fori_loop(0, nch // GRP, grp, 0)
            plsc.subcore_barrier()
            pltpu.sync_copy(acc.at[pl.ds(s * RPT, RPT)],
                            out_hbm.at[c].at[p].at[pl.ds(s * RPT, RPT)])

    return pl.kernel(
        body,
        mesh=mesh,
        compiler_params=pltpu.CompilerParams(use_tc_tiling_on_sc=False),
        out_type=jax.ShapeDtypeStruct((NCORE, npan, NP, d), jnp.float32),
        scratch_types=[
            pltpu.VMEM((GRP, CHUNK), jnp.int32),
            pltpu.VMEM((GRP, CHUNK), jnp.int32),
            pltpu.VMEM((CHUNK, d), jnp.float32),
            pltpu.VMEM((CHUNK, d), jnp.float32),
            pltpu.VMEM_SHARED((NP, d), jnp.float32),
            pltpu.SemaphoreType.DMA,
            pltpu.SemaphoreType.DMA,
            pltpu.SemaphoreType.DMA,
            pltpu.SemaphoreType.DMA,
        ],
    )


@functools.lru_cache(maxsize=None)
def _make_combine(d: int, relu: bool, post_scale: bool):
    def body(a_ref, u_ref, dv_ref, b_ref, o_ref):
        ssum = a_ref[0] + a_ref[1] + u_ref[...]
        y = dv_ref[...] * ssum + b_ref[...]
        if relu:
            y = jnp.maximum(y, 0.0)
        if post_scale:
            y = dv_ref[...] * y
        o_ref[...] = y

    return pl.pallas_call(
        body,
        grid=(NP // ROWBLK,),
        in_specs=[
            pl.BlockSpec((NCORE, ROWBLK, d), lambda i: (0, i, 0)),
            pl.BlockSpec((ROWBLK, d), lambda i: (i, 0)),
            pl.BlockSpec((ROWBLK, 1), lambda i: (i, 0)),
            pl.BlockSpec((1, d), lambda i: (0, 0)),
        ],
        out_specs=pl.BlockSpec((ROWBLK, d), lambda i: (i, 0)),
        out_shape=jax.ShapeDtypeStruct((NP, d), jnp.float32),
    )


@functools.lru_cache(maxsize=None)
def _make_matscale(k: int, d: int, relu: bool, scale: bool):
    def body(h_ref, w_ref, b_ref, dv_ref, o_ref):
        z = jnp.dot(h_ref[...], w_ref[...], preferred_element_type=jnp.float32)
        z = z + b_ref[...]
        if relu:
            z = jnp.maximum(z, 0.0)
        if scale:
            z = dv_ref[...] * z
        o_ref[...] = z

    return pl.pallas_call(
        body,
        grid=(NP // ROWBLK,),
        in_specs=[
            pl.BlockSpec((ROWBLK, k), lambda i: (i, 0)),
            pl.BlockSpec((k, d), lambda i: (0, 0)),
            pl.BlockSpec((1, d), lambda i: (0, 0)),
            pl.BlockSpec((ROWBLK, 1), lambda i: (i, 0)),
        ],
        out_specs=pl.BlockSpec((ROWBLK, d), lambda i: (i, 0)),
        out_shape=jax.ShapeDtypeStruct((NP, d), jnp.float32),
    )


def _dinv_body(dp_ref, o_ref):
    deg = dp_ref[0, :, 0:1] + dp_ref[1, :, 0:1] + 1.0
    o_ref[...] = lax.rsqrt(deg)


_dinv_call = None


def _make_dinv():
    global _dinv_call
    if _dinv_call is None:
        _dinv_call = pl.pallas_call(
            _dinv_body,
            grid=(NP // ROWBLK,),
            in_specs=[pl.BlockSpec((NCORE, ROWBLK, 16), lambda i: (0, i, 0))],
            out_specs=pl.BlockSpec((ROWBLK, 1), lambda i: (i, 0)),
            out_shape=jax.ShapeDtypeStruct((NP, 1), jnp.float32),
        )
    return _dinv_call


def kernel(x, edge_index, W0, b0, W1, b1, W2, b2, W3, b3, W4, b4, W5, b5,
           W6, b6, W7, b7):
    E = edge_index.shape[1]
    ept = -(-E // (NTILE * GRP * CHUNK)) * GRP * CHUNK
    nch = ept // CHUNK
    epad = ept * NTILE - E

    src = jnp.concatenate([edge_index[0], jnp.zeros((epad,), jnp.int32)])
    dst = jnp.concatenate([edge_index[1], jnp.full((epad,), NP - 1, jnp.int32)])
    src_r = src.reshape(NTILE, nch, CHUNK)
    dst_r = dst.reshape(NTILE, nch, CHUNK)

    def agg(z3, d):
        npan = z3.shape[0]
        zeros = jnp.zeros((RPT, d), jnp.float32)
        return _make_agg(nch, npan, d)(src_r, dst_r, z3, zeros)

    def agg1(z, d):
        return agg(z[None], d)[:, 0]

    def combine(a, u, dinv, b, d, relu, post_scale=False):
        bb = jnp.zeros((1, d), jnp.float32) if b is None else b.reshape(1, d)
        return _make_combine(d, relu, post_scale)(a, u, dinv, bb)

    def matscale(h, W, b, dinv, relu, scale):
        k, d = W.shape
        bb = jnp.zeros((1, d), jnp.float32) if b is None else b.reshape(1, d)
        return _make_matscale(k, d, relu, scale)(h, W, bb, dinv)

    ones16 = jnp.ones((NP, 16), jnp.float32)
    degp = agg1(ones16, 16)
    dinv = _make_dinv()(degp)

    xp = jnp.pad(x, ((0, NP - N), (0, 0)))

    u0 = matscale(xp, W0, None, dinv, relu=False, scale=True)
    a0 = agg1(u0, 64)
    h1 = combine(a0, u0, dinv, b0, 64, relu=True)
    u1 = matscale(h1, W1, None, dinv, relu=False, scale=True)
    a1 = agg1(u1, 64)
    h2 = combine(a1, u1, dinv, b1, 64, relu=True)
    u2 = matscale(h2, W2, None, dinv, relu=False, scale=True)
    a2 = agg1(u2, 64)
    u3 = combine(a2, u2, dinv, b2, 64, relu=True, post_scale=True)
    a3 = agg1(u3, 64)
    q3 = combine(a3, u3, dinv, None, 64, relu=False)
    u4 = matscale(q3, W3, b3, dinv, relu=True, scale=True)
    a4 = agg1(u4, 128)
    q4 = combine(a4, u4, dinv, None, 128, relu=False)
    h5 = matscale(q4, W4, b4, dinv, relu=True, scale=False)
    u5 = matscale(h5, W5, None, dinv, relu=False, scale=True)
    u5p3 = u5.reshape(NP, 4, 128).transpose(1, 0, 2)
    a5 = agg(u5p3, 128)
    h6p = [combine(a5[:, p], u5p3[p], dinv,
                   lax.slice_in_dim(b5, p * 128, (p + 1) * 128), 128, relu=True)
           for p in range(4)]
    h6 = jnp.concatenate(h6p, axis=1)
    u6 = matscale(h6, W6, None, dinv, relu=False, scale=True)
    u6p3 = u6.reshape(NP, 2, 128).transpose(1, 0, 2)
    a6 = agg(u6p3, 128)
    h7p = [combine(a6[:, p], u6p3[p], dinv,
                   lax.slice_in_dim(b6, p * 128, (p + 1) * 128), 128, relu=True)
           for p in range(2)]
    h7 = jnp.concatenate(h7p, axis=1)
    W7p = jnp.pad(W7, ((0, 0), (0, 14)))
    b7p = jnp.pad(b7, ((0, 14),))
    u7 = matscale(h7, W7p, None, dinv, relu=False, scale=True)
    a7 = agg1(u7, 16)
    outp = combine(a7, u7, dinv, b7p, 16, relu=False)
    return outp[:N, :2]
